# baseline (speedup 1.0000x reference)
import functools

import jax
import jax.numpy as jnp
from jax import lax
from jax.experimental import pallas as pl
from jax.experimental.pallas import tpu as pltpu
from jax.experimental.pallas import tpu_sc as plsc

D_MODEL = 128
SEQ = 50
LANES = 16
NUM_WORKERS = 32
CHUNK = 200


def _positional_encoding(seq, d_model):
    pos = jnp.arange(seq, dtype=jnp.float32)[:, None]
    i = jnp.arange(0, d_model, 2, dtype=jnp.float32)
    div = jnp.exp(-i * (jnp.log(10000.0) / d_model))
    ang = pos * div
    pe = jnp.zeros((seq, d_model), dtype=jnp.float32)
    pe = pe.at[:, 0::2].set(jnp.sin(ang))
    pe = pe.at[:, 1::2].set(jnp.cos(ang))
    return pe


def _make_sc_kernel(n_rows, n_chunks):
    mesh = plsc.VectorSubcoreMesh(core_axis_name="c", subcore_axis_name="s")
    rows_per_w = n_rows // NUM_WORKERS
    n_dreg = D_MODEL // LANES

    @functools.partial(
        pl.kernel,
        mesh=mesh,
        out_type=jax.ShapeDtypeStruct((n_rows, D_MODEL), jnp.float32),
        scratch_types=[
            pltpu.VMEM((CHUNK,), jnp.int32),
            pltpu.VMEM((SEQ, D_MODEL), jnp.float32),
            pltpu.VMEM((CHUNK, D_MODEL), jnp.float32),
            pltpu.SemaphoreType.DMA,
        ],
    )
    def sc_embed(x_hbm, tab_hbm, pe_hbm, out_hbm, idx_v, pe_v, buf_v, sem):
        cid = lax.axis_index("c")
        sid = lax.axis_index("s")
        w = sid * 2 + cid
        pltpu.sync_copy(pe_hbm, pe_v)
        base = w * rows_per_w

        def chunk_body(c, carry):
            pltpu.sync_copy(x_hbm.at[w, c], idx_v)
            pltpu.async_copy(tab_hbm.at[idx_v], buf_v, sem).wait()

            def pe_body(s, carry2):
                for j in range(CHUNK // SEQ):
                    r = j * SEQ + s
                    for d in range(n_dreg):
                        sl = pl.ds(d * LANES, LANES)
                        buf_v[r, sl] = buf_v[r, sl] + pe_v[s, sl]
                return carry2

            lax.fori_loop(0, SEQ, pe_body, 0)
            pltpu.sync_copy(buf_v, out_hbm.at[pl.ds(base + c * CHUNK, CHUNK)])
            return carry

        lax.fori_loop(0, n_chunks, chunk_body, 0)

    return sc_embed


def kernel(x, tok_table):
    batch, seq = x.shape
    assert seq == SEQ
    n_rows = batch * seq
    assert n_rows % (NUM_WORKERS * CHUNK) == 0
    n_chunks = n_rows // (NUM_WORKERS * CHUNK)
    x_flat = x.astype(jnp.int32).reshape(NUM_WORKERS, n_chunks, CHUNK)
    pe = _positional_encoding(SEQ, D_MODEL)
    sc_embed = _make_sc_kernel(n_rows, n_chunks)
    out = sc_embed(x_flat, tok_table, pe)
    return out.reshape(batch, seq, D_MODEL)
